# Initial kernel scaffold; baseline (speedup 1.0000x reference)
import jax
import jax.numpy as jnp
from jax.experimental import pallas as pl


def kernel(xyz_A, feat_A, params):
    raise NotImplementedError("write your pallas kernel here")


# speedup vs baseline: 1.5190x; 1.5190x over previous
from functools import partial

import jax
import jax.numpy as jnp
from jax.experimental import pallas as pl

_PREC = jax.lax.Precision.HIGHEST


def _coldot(a, b):
    return jax.lax.dot_general(a, b, (((1,), (0,)), ((), ())),
                               preferred_element_type=jnp.float32,
                               precision=_PREC)


def _sa_group_kernel(xyzT_ref, xyzn_ref, feat_ref, g_ref, nxyz_ref,
                     *, n, npoint, k, r2):
    x = xyzT_ref[0]
    xn = xyzn_ref[0]
    f = feat_ref[0]
    lane = jax.lax.broadcasted_iota(jnp.int32, (1, n), 1)
    lane_np = jax.lax.broadcasted_iota(jnp.int32, (1, npoint), 1)
    sub = jax.lax.broadcasted_iota(jnp.int32, (n, 1), 0)

    def body(i, carry):
        dists, far, new_t = carry
        oh = (lane == far).astype(jnp.float32)
        cent = jnp.sum(x * oh, axis=1, keepdims=True)
        new_t = jnp.where(lane_np == i, cent, new_t)
        d = jnp.sum((x - cent) ** 2, axis=0, keepdims=True)
        dists = jnp.minimum(dists, d)
        m = jnp.max(dists)
        far = jnp.min(jnp.where(dists == m, lane, n)).astype(jnp.int32)
        return dists, far, new_t

    dists0 = jnp.full((1, n), 1e10, jnp.float32)
    _, _, new_t = jax.lax.fori_loop(
        0, npoint, body,
        (dists0, jnp.int32(0), jnp.zeros((3, npoint), jnp.float32)))
    nxyz_ref[0] = new_t

    d2 = ((xn[:, 0:1] - new_t[0:1, :]) ** 2
          + (xn[:, 1:2] - new_t[1:2, :]) ** 2
          + (xn[:, 2:3] - new_t[2:3, :]) ** 2)

    keyv = jnp.where(d2 < r2, sub, n)
    cols = []
    for _ in range(k):
        mj = jnp.min(keyv, axis=0, keepdims=True)
        cols.append(mj)
        keyv = jnp.where(keyv == mj, n, keyv)
    first = cols[0]
    for j in range(k):
        cj = jnp.where(cols[j] < n, cols[j], first)
        ohj = (sub == cj).astype(jnp.float32)
        gx = _coldot(x, ohj) - new_t
        gf = _coldot(f, ohj)
        g_ref[0, :, j, :] = jnp.concatenate([gx, gf], axis=0)


def _sa_group(xyzT, xyzn, feat, npoint, k, r2):
    b, _, n = xyzT.shape
    c = feat.shape[1]
    fn = partial(_sa_group_kernel, n=n, npoint=npoint, k=k, r2=r2)
    return pl.pallas_call(
        fn,
        grid=(b,),
        in_specs=[pl.BlockSpec((1, 3, n), lambda i: (i, 0, 0)),
                  pl.BlockSpec((1, n, 3), lambda i: (i, 0, 0)),
                  pl.BlockSpec((1, c, n), lambda i: (i, 0, 0))],
        out_specs=[pl.BlockSpec((1, 3 + c, k, npoint), lambda i: (i, 0, 0, 0)),
                   pl.BlockSpec((1, 3, npoint), lambda i: (i, 0, 0))],
        out_shape=[jax.ShapeDtypeStruct((b, 3 + c, k, npoint), jnp.float32),
                   jax.ShapeDtypeStruct((b, 3, npoint), jnp.float32)],
    )(xyzT, xyzn, feat)


def _bn_layer(x, w_ref, b_ref, g_ref, be_ref):
    h = jax.lax.dot_general(w_ref[...], x, (((1,), (0,)), ((), ())),
                            preferred_element_type=jnp.float32) + b_ref[...]
    m = jnp.mean(h, axis=1, keepdims=True)
    v = jnp.mean((h - m) ** 2, axis=1, keepdims=True)
    h = g_ref[...] * (h - m) / jnp.sqrt(v + 1e-5) + be_ref[...]
    return jnp.maximum(h, 0.0)


def _mlp_kernel(x_ref, w1, b1, g1, be1, w2, b2, g2, be2, out_ref, *, npool):
    h = _bn_layer(x_ref[...], w1, b1, g1, be1)
    h = _bn_layer(h, w2, b2, g2, be2)
    if npool:
        width = h.shape[1] // npool
        r = h[:, 0:width]
        for j in range(1, npool):
            r = jnp.maximum(r, h[:, j * width:(j + 1) * width])
        out_ref[...] = r
    else:
        out_ref[...] = h


def _mlp(x, layers, npool):
    (w1, b1, g1, be1), (w2, b2, g2, be2) = layers
    c1, c2 = w1.shape[0], w2.shape[0]
    out_l = x.shape[1] // npool if npool else x.shape[1]
    return pl.pallas_call(
        partial(_mlp_kernel, npool=npool),
        out_shape=jax.ShapeDtypeStruct((c2, out_l), jnp.float32),
    )(x, w1, b1.reshape(c1, 1), g1.reshape(c1, 1), be1.reshape(c1, 1),
      w2, b2.reshape(c2, 1), g2.reshape(c2, 1), be2.reshape(c2, 1))


def _pool_kernel(x_ref, o_ref):
    o_ref[...] = jnp.max(x_ref[...], axis=2, keepdims=True)


def kernel(xyz_A, feat_A, params):
    p1, p2, p3 = params
    b = xyz_A.shape[0]
    xyzT = jnp.transpose(xyz_A, (0, 2, 1))

    g1, xyz1T = _sa_group(xyzT, xyz_A, feat_A, 128, 4, 0.25 * 0.25)
    x1 = jnp.transpose(g1, (1, 2, 0, 3)).reshape(6, 4 * b * 128)
    f1 = _mlp(x1, p1, npool=4)
    f1b = jnp.transpose(f1.reshape(128, b, 128), (1, 0, 2))

    xyz1n = jnp.transpose(xyz1T, (0, 2, 1))
    g2, xyz2T = _sa_group(xyz1T, xyz1n, f1b, 32, 4, 0.4 * 0.4)
    x2 = jnp.transpose(g2, (1, 2, 0, 3)).reshape(131, 4 * b * 32)
    f2 = _mlp(x2, p2, npool=4)

    x3 = jnp.concatenate(
        [jnp.transpose(xyz2T, (1, 0, 2)).reshape(3, b * 32), f2], axis=0)
    h3 = _mlp(x3, p3, npool=0)
    h3r = jnp.transpose(h3.reshape(512, b, 32), (1, 0, 2))
    return pl.pallas_call(
        _pool_kernel,
        out_shape=jax.ShapeDtypeStruct((b, 512, 1), jnp.float32),
    )(h3r)


# speedup vs baseline: 7.5772x; 4.9882x over previous
from functools import partial

import jax
import jax.numpy as jnp
from jax.experimental import pallas as pl


def _fps_kernel(xc_ref, nt_ref, *, n, npoint, nb):
    x0, x1, x2 = xc_ref[0], xc_ref[1], xc_ref[2]
    lane = jax.lax.broadcasted_iota(jnp.int32, (1, n), 1)
    lane_np = jax.lax.broadcasted_iota(jnp.int32, (1, npoint), 1)

    def body(i, carry):
        dists, far, nt0, nt1, nt2 = carry
        oh = (lane == far).astype(jnp.float32)
        c0 = jnp.sum(x0 * oh, axis=1, keepdims=True)
        c1 = jnp.sum(x1 * oh, axis=1, keepdims=True)
        c2 = jnp.sum(x2 * oh, axis=1, keepdims=True)
        nt0 = jnp.where(lane_np == i, c0, nt0)
        nt1 = jnp.where(lane_np == i, c1, nt1)
        nt2 = jnp.where(lane_np == i, c2, nt2)
        d = (x0 - c0) ** 2 + (x1 - c1) ** 2 + (x2 - c2) ** 2
        dists = jnp.minimum(dists, d)
        m = jnp.max(dists, axis=1, keepdims=True)
        far = jnp.min(jnp.where(dists == m, lane, n), axis=1, keepdims=True)
        return dists, far, nt0, nt1, nt2

    z = jnp.zeros((nb, npoint), jnp.float32)
    _, _, nt0, nt1, nt2 = jax.lax.fori_loop(
        0, npoint, body,
        (jnp.full((nb, n), 1e10, jnp.float32),
         jnp.zeros((nb, 1), jnp.int32), z, z, z))
    nt_ref[0] = nt0
    nt_ref[1] = nt1
    nt_ref[2] = nt2


def _fps(xc, npoint):
    _, nb, n = xc.shape
    return pl.pallas_call(
        partial(_fps_kernel, n=n, npoint=npoint, nb=nb),
        out_shape=jax.ShapeDtypeStruct((3, nb, npoint), jnp.float32),
    )(xc)


def _group_kernel(xf_ref, xn_ref, nt_ref, g_ref, *, n, npoint, k, r2):
    xf = xf_ref[0]
    xn = xn_ref[0]
    nt = nt_ref[0]
    sub = jax.lax.broadcasted_iota(jnp.int32, (n, 1), 0)

    d2 = ((xn[:, 0:1] - nt[0:1, :]) ** 2
          + (xn[:, 1:2] - nt[1:2, :]) ** 2
          + (xn[:, 2:3] - nt[2:3, :]) ** 2)

    keyv = jnp.where(d2 < r2, sub, n)
    cols = []
    for _ in range(k):
        mj = jnp.min(keyv, axis=0, keepdims=True)
        cols.append(mj)
        keyv = jnp.where(keyv == mj, n, keyv)
    first = cols[0]
    cj = jnp.concatenate(
        [jnp.where(c < n, c, first) for c in cols], axis=1)
    oh = (sub == cj).astype(jnp.float32)
    g = jax.lax.dot_general(xf, oh, (((1,), (0,)), ((), ())),
                            preferred_element_type=jnp.float32,
                            precision=jax.lax.Precision.HIGHEST)
    ntk = jnp.concatenate([nt] * k, axis=1)
    zc = jnp.zeros((xf.shape[0] - 3, k * npoint), jnp.float32)
    g_ref[0] = g - jnp.concatenate([ntk, zc], axis=0)


def _group(xf, xn, ntb, npoint, k, r2):
    b, cf, n = xf.shape
    fn = partial(_group_kernel, n=n, npoint=npoint, k=k, r2=r2)
    return pl.pallas_call(
        fn,
        grid=(b,),
        in_specs=[pl.BlockSpec((1, cf, n), lambda i: (i, 0, 0)),
                  pl.BlockSpec((1, n, 3), lambda i: (i, 0, 0)),
                  pl.BlockSpec((1, 3, npoint), lambda i: (i, 0, 0))],
        out_specs=pl.BlockSpec((1, cf, k * npoint), lambda i: (i, 0, 0)),
        out_shape=jax.ShapeDtypeStruct((b, cf, k * npoint), jnp.float32),
    )(xf, xn, ntb)


def _bn_layer(x, w_ref, b_ref, g_ref, be_ref):
    h = jax.lax.dot_general(w_ref[...], x, (((1,), (0,)), ((), ())),
                            preferred_element_type=jnp.float32) + b_ref[...]
    m = jnp.mean(h, axis=1, keepdims=True)
    v = jnp.mean((h - m) ** 2, axis=1, keepdims=True)
    h = g_ref[...] * (h - m) / jnp.sqrt(v + 1e-5) + be_ref[...]
    return jnp.maximum(h, 0.0)


def _mlp_kernel(x_ref, w1, b1, g1, be1, w2, b2, g2, be2, out_ref, *, npool):
    h = _bn_layer(x_ref[...], w1, b1, g1, be1)
    h = _bn_layer(h, w2, b2, g2, be2)
    if npool:
        width = h.shape[1] // npool
        r = h[:, 0:width]
        for j in range(1, npool):
            r = jnp.maximum(r, h[:, j * width:(j + 1) * width])
        out_ref[...] = r
    else:
        out_ref[...] = h


def _mlp(x, layers, npool):
    (w1, b1, g1, be1), (w2, b2, g2, be2) = layers
    c1, c2 = w1.shape[0], w2.shape[0]
    out_l = x.shape[1] // npool if npool else x.shape[1]
    return pl.pallas_call(
        partial(_mlp_kernel, npool=npool),
        out_shape=jax.ShapeDtypeStruct((c2, out_l), jnp.float32),
    )(x, w1, b1.reshape(c1, 1), g1.reshape(c1, 1), be1.reshape(c1, 1),
      w2, b2.reshape(c2, 1), g2.reshape(c2, 1), be2.reshape(c2, 1))


def _pool_kernel(x_ref, o_ref):
    o_ref[...] = jnp.max(x_ref[...], axis=2, keepdims=True)


def kernel(xyz_A, feat_A, params):
    p1, p2, p3 = params
    b = xyz_A.shape[0]
    xc = jnp.transpose(xyz_A, (2, 0, 1))
    xyzT = jnp.transpose(xyz_A, (0, 2, 1))

    nt1 = _fps(xc, 128)
    xf1 = jnp.concatenate([xyzT, feat_A], axis=1)
    nt1b = jnp.transpose(nt1, (1, 0, 2))
    g1 = _group(xf1, xyz_A, nt1b, 128, 4, 0.25 * 0.25)
    x1 = jnp.transpose(g1.reshape(b, 6, 4, 128),
                       (1, 2, 0, 3)).reshape(6, 4 * b * 128)
    f1 = _mlp(x1, p1, npool=4)
    f1b = jnp.transpose(f1.reshape(128, b, 128), (1, 0, 2))

    nt2 = _fps(nt1, 32)
    xf2 = jnp.concatenate([nt1b, f1b], axis=1)
    xn2 = jnp.transpose(nt1, (1, 2, 0))
    nt2b = jnp.transpose(nt2, (1, 0, 2))
    g2 = _group(xf2, xn2, nt2b, 32, 4, 0.4 * 0.4)
    x2 = jnp.transpose(g2.reshape(b, 131, 4, 32),
                       (1, 2, 0, 3)).reshape(131, 4 * b * 32)
    f2 = _mlp(x2, p2, npool=4)

    x3 = jnp.concatenate([nt2.reshape(3, b * 32), f2], axis=0)
    h3 = _mlp(x3, p3, npool=0)
    h3r = jnp.transpose(h3.reshape(512, b, 32), (1, 0, 2))
    return pl.pallas_call(
        _pool_kernel,
        out_shape=jax.ShapeDtypeStruct((b, 512, 1), jnp.float32),
    )(h3r)


# speedup vs baseline: 7.9310x; 1.0467x over previous
from functools import partial

import jax
import jax.numpy as jnp
from jax.experimental import pallas as pl


def _fps_kernel(xc_ref, nt_ref, *, n, npoint, nb):
    x0, x1, x2 = xc_ref[0], xc_ref[1], xc_ref[2]
    lane = jax.lax.broadcasted_iota(jnp.int32, (1, n), 1)
    lane_np = jax.lax.broadcasted_iota(jnp.int32, (1, npoint), 1)

    def body(i, carry):
        dists, far, nt0, nt1, nt2 = carry
        oh = (lane == far).astype(jnp.float32)
        c0 = jnp.sum(x0 * oh, axis=1, keepdims=True)
        c1 = jnp.sum(x1 * oh, axis=1, keepdims=True)
        c2 = jnp.sum(x2 * oh, axis=1, keepdims=True)
        nt0 = jnp.where(lane_np == i, c0, nt0)
        nt1 = jnp.where(lane_np == i, c1, nt1)
        nt2 = jnp.where(lane_np == i, c2, nt2)
        d = (x0 - c0) ** 2 + (x1 - c1) ** 2 + (x2 - c2) ** 2
        dists = jnp.minimum(dists, d)
        m = jnp.max(dists, axis=1, keepdims=True)
        far = jnp.min(jnp.where(dists == m, lane, n), axis=1, keepdims=True)
        return dists, far, nt0, nt1, nt2

    z = jnp.zeros((nb, npoint), jnp.float32)
    _, _, nt0, nt1, nt2 = jax.lax.fori_loop(
        0, npoint, body,
        (jnp.full((nb, n), 1e10, jnp.float32),
         jnp.zeros((nb, 1), jnp.int32), z, z, z))
    nt_ref[0] = nt0
    nt_ref[1] = nt1
    nt_ref[2] = nt2


def _fps(xc, npoint):
    _, nb, n = xc.shape
    return pl.pallas_call(
        partial(_fps_kernel, n=n, npoint=npoint, nb=nb),
        out_shape=jax.ShapeDtypeStruct((3, nb, npoint), jnp.float32),
    )(xc)


def _group_kernel(xyzT_ref, feat_ref, xn_ref, nt_ref, g_ref,
                  *, n, npoint, k, r2, feat3d):
    xyzb = xyzT_ref[0]
    f = feat_ref[0] if feat3d else feat_ref[...]
    xn = xn_ref[0]
    nt = nt_ref[0]
    sub = jax.lax.broadcasted_iota(jnp.int32, (n, 1), 0)

    d2 = ((xn[:, 0:1] - nt[0:1, :]) ** 2
          + (xn[:, 1:2] - nt[1:2, :]) ** 2
          + (xn[:, 2:3] - nt[2:3, :]) ** 2)

    keyv = jnp.where(d2 < r2, sub, n)
    cols = []
    for _ in range(k):
        mj = jnp.min(keyv, axis=0, keepdims=True)
        cols.append(mj)
        keyv = jnp.where(keyv == mj, n, keyv)
    first = cols[0]
    cj = jnp.concatenate(
        [jnp.where(c < n, c, first) for c in cols], axis=1)
    oh = (sub == cj).astype(jnp.float32)
    xf = jnp.concatenate([xyzb, f], axis=0)
    g = jax.lax.dot_general(xf, oh, (((1,), (0,)), ((), ())),
                            preferred_element_type=jnp.float32,
                            precision=jax.lax.Precision.HIGHEST)
    ntk = jnp.concatenate([nt] * k, axis=1)
    zc = jnp.zeros((f.shape[0], k * npoint), jnp.float32)
    g_ref[0] = g - jnp.concatenate([ntk, zc], axis=0)


def _group(xyzT, feat, xn, ntb, npoint, k, r2):
    b, _, n = xyzT.shape
    feat3d = feat.ndim == 3
    c = feat.shape[1] if feat3d else feat.shape[0]
    if feat3d:
        feat_spec = pl.BlockSpec((1, c, n), lambda i: (i, 0, 0))
    else:
        feat_spec = pl.BlockSpec((c, n), lambda i: (0, i))
    fn = partial(_group_kernel, n=n, npoint=npoint, k=k, r2=r2, feat3d=feat3d)
    return pl.pallas_call(
        fn,
        grid=(b,),
        in_specs=[pl.BlockSpec((1, 3, n), lambda i: (i, 0, 0)),
                  feat_spec,
                  pl.BlockSpec((1, n, 3), lambda i: (i, 0, 0)),
                  pl.BlockSpec((1, 3, npoint), lambda i: (i, 0, 0))],
        out_specs=pl.BlockSpec((1, 3 + c, k * npoint), lambda i: (i, 0, 0)),
        out_shape=jax.ShapeDtypeStruct((b, 3 + c, k * npoint), jnp.float32),
    )(xyzT, feat, xn, ntb)


def _bn_layer(x, w_ref, b_ref, g_ref, be_ref):
    h = jax.lax.dot_general(w_ref[...], x, (((1,), (0,)), ((), ())),
                            preferred_element_type=jnp.float32) + b_ref[...]
    m = jnp.mean(h, axis=1, keepdims=True)
    v = jnp.mean((h - m) ** 2, axis=1, keepdims=True)
    h = g_ref[...] * (h - m) / jnp.sqrt(v + 1e-5) + be_ref[...]
    return jnp.maximum(h, 0.0)


def _mlp_kernel(x_ref, w1, b1, g1, be1, w2, b2, g2, be2, out_ref,
                *, npool, treepool):
    h = _bn_layer(x_ref[...], w1, b1, g1, be1)
    h = _bn_layer(h, w2, b2, g2, be2)
    if npool:
        width = h.shape[1] // npool
        r = h[:, 0:width]
        for j in range(1, npool):
            r = jnp.maximum(r, h[:, j * width:(j + 1) * width])
        h = r
    if treepool:
        w = h.shape[1]
        while w > treepool:
            w //= 2
            h = jnp.maximum(h[:, 0:w], h[:, w:2 * w])
    out_ref[...] = h


def _mlp(x, layers, npool, treepool=0):
    (w1, b1, g1, be1), (w2, b2, g2, be2) = layers
    c1, c2 = w1.shape[0], w2.shape[0]
    out_l = x.shape[1] // npool if npool else x.shape[1]
    if treepool:
        out_l = treepool
    return pl.pallas_call(
        partial(_mlp_kernel, npool=npool, treepool=treepool),
        out_shape=jax.ShapeDtypeStruct((c2, out_l), jnp.float32),
    )(x, w1, b1.reshape(c1, 1), g1.reshape(c1, 1), be1.reshape(c1, 1),
      w2, b2.reshape(c2, 1), g2.reshape(c2, 1), be2.reshape(c2, 1))


def kernel(xyz_A, feat_A, params):
    p1, p2, p3 = params
    b = xyz_A.shape[0]
    xc = jnp.transpose(xyz_A, (2, 0, 1))
    xyzT = jnp.transpose(xyz_A, (0, 2, 1))

    nt1 = _fps(xc, 128)
    nt1b = jnp.transpose(nt1, (1, 0, 2))
    g1 = _group(xyzT, feat_A, xyz_A, nt1b, 128, 4, 0.25 * 0.25)
    x1 = jnp.transpose(g1.reshape(b, 6, 4, 128),
                       (1, 2, 0, 3)).reshape(6, 4 * b * 128)
    f1 = _mlp(x1, p1, npool=4)

    nt2 = _fps(nt1, 32)
    xn2 = jnp.transpose(nt1, (1, 2, 0))
    nt2b = jnp.transpose(nt2, (1, 0, 2))
    g2 = _group(nt1b, f1, xn2, nt2b, 32, 4, 0.4 * 0.4)
    x2 = jnp.transpose(g2.reshape(b, 131, 4, 32),
                       (1, 2, 3, 0)).reshape(131, 4 * b * 32)
    f2 = _mlp(x2, p2, npool=4)

    x3 = jnp.concatenate(
        [jnp.transpose(nt2, (0, 2, 1)).reshape(3, b * 32), f2], axis=0)
    h3 = _mlp(x3, p3, npool=0, treepool=b)
    return jnp.transpose(h3)[:, :, None]


# speedup vs baseline: 9.9300x; 1.2520x over previous
from functools import partial

import jax
import jax.numpy as jnp
from jax.experimental import pallas as pl


def _fps_kernel(xc_ref, nt_ref, idx_ref, *, n, npoint, nb, r2):
    x0, x1, x2 = xc_ref[0], xc_ref[1], xc_ref[2]
    lane = jax.lax.broadcasted_iota(jnp.int32, (1, n), 1)
    lane_np = jax.lax.broadcasted_iota(jnp.int32, (1, npoint), 1)

    def body(i, carry):
        dists, far, nt0, nt1, nt2, i0, i1, i2, i3 = carry
        oh = (lane == far).astype(jnp.float32)
        c0 = jnp.sum(x0 * oh, axis=1, keepdims=True)
        c1 = jnp.sum(x1 * oh, axis=1, keepdims=True)
        c2 = jnp.sum(x2 * oh, axis=1, keepdims=True)
        nt0 = jnp.where(lane_np == i, c0, nt0)
        nt1 = jnp.where(lane_np == i, c1, nt1)
        nt2 = jnp.where(lane_np == i, c2, nt2)
        d = (x0 - c0) ** 2 + (x1 - c1) ** 2 + (x2 - c2) ** 2

        keyv = jnp.where(d < r2, lane, n)
        m0 = jnp.min(keyv, axis=1, keepdims=True)
        keyv = jnp.where(keyv == m0, n, keyv)
        m1 = jnp.min(keyv, axis=1, keepdims=True)
        keyv = jnp.where(keyv == m1, n, keyv)
        m2 = jnp.min(keyv, axis=1, keepdims=True)
        keyv = jnp.where(keyv == m2, n, keyv)
        m3 = jnp.min(keyv, axis=1, keepdims=True)
        m1 = jnp.where(m1 < n, m1, m0)
        m2 = jnp.where(m2 < n, m2, m0)
        m3 = jnp.where(m3 < n, m3, m0)
        i0 = jnp.where(lane_np == i, m0, i0)
        i1 = jnp.where(lane_np == i, m1, i1)
        i2 = jnp.where(lane_np == i, m2, i2)
        i3 = jnp.where(lane_np == i, m3, i3)

        dists = jnp.minimum(dists, d)
        m = jnp.max(dists, axis=1, keepdims=True)
        far = jnp.min(jnp.where(dists == m, lane, n), axis=1, keepdims=True)
        return dists, far, nt0, nt1, nt2, i0, i1, i2, i3

    z = jnp.zeros((nb, npoint), jnp.float32)
    zi = jnp.zeros((nb, npoint), jnp.int32)
    _, _, nt0, nt1, nt2, i0, i1, i2, i3 = jax.lax.fori_loop(
        0, npoint, body,
        (jnp.full((nb, n), 1e10, jnp.float32),
         jnp.zeros((nb, 1), jnp.int32), z, z, z, zi, zi, zi, zi))
    nt_ref[0] = nt0
    nt_ref[1] = nt1
    nt_ref[2] = nt2
    idx_ref[0] = i0
    idx_ref[1] = i1
    idx_ref[2] = i2
    idx_ref[3] = i3


def _fps(xc, npoint, r2):
    _, nb, n = xc.shape
    return pl.pallas_call(
        partial(_fps_kernel, n=n, npoint=npoint, nb=nb, r2=r2),
        out_shape=[jax.ShapeDtypeStruct((3, nb, npoint), jnp.float32),
                   jax.ShapeDtypeStruct((4, nb, npoint), jnp.int32)],
    )(xc)


def _split_bf16(x):
    hi = x.astype(jnp.bfloat16)
    mid = (x - hi.astype(jnp.float32)).astype(jnp.bfloat16)
    return hi, mid


def _bfdot(a, b):
    return jax.lax.dot_general(a, b, (((1,), (0,)), ((), ())),
                               preferred_element_type=jnp.float32)


def _gather_kernel(xfr_ref, idx_ref, nt_ref, g_ref, *, nq, c, npoint, k):
    xfr = xfr_ref[...] if xfr_ref.ndim == 2 else xfr_ref[0]
    idxb = idx_ref[0]
    nt = nt_ref[0]
    cj = jnp.concatenate([idxb[j:j + 1, :] for j in range(k)], axis=1)
    sub = jax.lax.broadcasted_iota(jnp.int32, (128, 1), 0)
    rj = jnp.bitwise_and(cj, 127)
    ohr = (sub == rj)
    hi, mid = _split_bf16(xfr)
    ohb = ohr.astype(jnp.bfloat16)
    z = _bfdot(hi, ohb) + _bfdot(mid, ohb)
    if nq > 1:
        qj = jnp.right_shift(cj, 7)
        g = z[0:c, :] * (qj == 0).astype(jnp.float32)
        for q in range(1, nq):
            g = g + z[q * c:(q + 1) * c, :] * (qj == q).astype(jnp.float32)
    else:
        g = z
    ntk = jnp.concatenate([nt] * k, axis=1)
    zc = jnp.zeros((c - 3, k * npoint), jnp.float32)
    g_ref[0] = g - jnp.concatenate([ntk, zc], axis=0)


def _bn_layer(x, w_ref, b_ref, g_ref, be_ref):
    h = jax.lax.dot_general(w_ref[...], x, (((1,), (0,)), ((), ())),
                            preferred_element_type=jnp.float32) + b_ref[...]
    m = jnp.mean(h, axis=1, keepdims=True)
    v = jnp.mean((h - m) ** 2, axis=1, keepdims=True)
    h = g_ref[...] * (h - m) / jnp.sqrt(v + 1e-5) + be_ref[...]
    return jnp.maximum(h, 0.0)


def _mlp_kernel(x_ref, w1, b1, g1, be1, w2, b2, g2, be2, out_ref,
                *, npool, treepool):
    h = _bn_layer(x_ref[...], w1, b1, g1, be1)
    h = _bn_layer(h, w2, b2, g2, be2)
    if npool:
        width = h.shape[1] // npool
        r = h[:, 0:width]
        for j in range(1, npool):
            r = jnp.maximum(r, h[:, j * width:(j + 1) * width])
        h = r
    if treepool:
        w = h.shape[1]
        while w > treepool:
            w //= 2
            h = jnp.maximum(h[:, 0:w], h[:, w:2 * w])
    out_ref[...] = h


def _mlp(x, layers, npool, treepool=0):
    (w1, b1, g1, be1), (w2, b2, g2, be2) = layers
    c1, c2 = w1.shape[0], w2.shape[0]
    out_l = x.shape[1] // npool if npool else x.shape[1]
    if treepool:
        out_l = treepool
    return pl.pallas_call(
        partial(_mlp_kernel, npool=npool, treepool=treepool),
        out_shape=jax.ShapeDtypeStruct((c2, out_l), jnp.float32),
    )(x, w1, b1.reshape(c1, 1), g1.reshape(c1, 1), be1.reshape(c1, 1),
      w2, b2.reshape(c2, 1), g2.reshape(c2, 1), be2.reshape(c2, 1))


def kernel(xyz_A, feat_A, params):
    p1, p2, p3 = params
    b = xyz_A.shape[0]
    xc = jnp.transpose(xyz_A, (2, 0, 1))
    xyzT = jnp.transpose(xyz_A, (0, 2, 1))

    nt1, idx1 = _fps(xc, 128, 0.25 * 0.25)
    nt1b = jnp.transpose(nt1, (1, 0, 2))
    xfr1 = jnp.concatenate(
        [jnp.transpose(xyzT.reshape(b, 3, 32, 128), (0, 2, 1, 3)),
         jnp.transpose(feat_A.reshape(b, 3, 32, 128), (0, 2, 1, 3))],
        axis=2).reshape(b, 192, 128)
    idx1b = jnp.transpose(idx1, (1, 0, 2))
    g1 = pl.pallas_call(
        partial(_gather_kernel, nq=32, c=6, npoint=128, k=4),
        grid=(b,),
        in_specs=[pl.BlockSpec((1, 192, 128), lambda i: (i, 0, 0)),
                  pl.BlockSpec((1, 4, 128), lambda i: (i, 0, 0)),
                  pl.BlockSpec((1, 3, 128), lambda i: (i, 0, 0))],
        out_specs=pl.BlockSpec((1, 6, 512), lambda i: (i, 0, 0)),
        out_shape=jax.ShapeDtypeStruct((b, 6, 512), jnp.float32),
    )(xfr1, idx1b, nt1b)
    x1 = jnp.transpose(g1.reshape(b, 6, 4, 128),
                       (1, 2, 0, 3)).reshape(6, 4 * b * 128)
    f1 = _mlp(x1, p1, npool=4)

    nt2, idx2 = _fps(nt1, 32, 0.4 * 0.4)
    nt2b = jnp.transpose(nt2, (1, 0, 2))
    idx2b = jnp.transpose(idx2, (1, 0, 2))

    def _gather2(ntb_ref, f_ref, idx_ref, nt_ref, g_ref):
        xfr = jnp.concatenate([ntb_ref[0], f_ref[...]], axis=0)
        _gather_body(xfr, idx_ref, nt_ref, g_ref)

    def _gather_body(xfr, idx_ref, nt_ref, g_ref):
        idxb = idx_ref[0]
        nt = nt_ref[0]
        cj = jnp.concatenate([idxb[j:j + 1, :] for j in range(4)], axis=1)
        sub = jax.lax.broadcasted_iota(jnp.int32, (128, 1), 0)
        ohb = (sub == cj).astype(jnp.bfloat16)
        hi, mid = _split_bf16(xfr)
        g = _bfdot(hi, ohb) + _bfdot(mid, ohb)
        ntk = jnp.concatenate([nt] * 4, axis=1)
        zc = jnp.zeros((128, g.shape[1]), jnp.float32)
        g_ref[0] = g - jnp.concatenate([ntk, zc], axis=0)

    g2 = pl.pallas_call(
        _gather2,
        grid=(b,),
        in_specs=[pl.BlockSpec((1, 3, 128), lambda i: (i, 0, 0)),
                  pl.BlockSpec((128, 128), lambda i: (0, i)),
                  pl.BlockSpec((1, 4, 32), lambda i: (i, 0, 0)),
                  pl.BlockSpec((1, 3, 32), lambda i: (i, 0, 0))],
        out_specs=pl.BlockSpec((1, 131, 128), lambda i: (i, 0, 0)),
        out_shape=jax.ShapeDtypeStruct((b, 131, 128), jnp.float32),
    )(nt1b, f1, idx2b, nt2b)
    x2 = jnp.transpose(g2.reshape(b, 131, 4, 32),
                       (1, 2, 3, 0)).reshape(131, 4 * b * 32)
    f2 = _mlp(x2, p2, npool=4)

    x3 = jnp.concatenate(
        [jnp.transpose(nt2, (0, 2, 1)).reshape(3, b * 32), f2], axis=0)
    h3 = _mlp(x3, p3, npool=0, treepool=b)
    return jnp.transpose(h3)[:, :, None]
